# baseline (speedup 1.0000x reference)
import functools

import jax
import jax.numpy as jnp
from jax import lax
from jax.experimental import pallas as pl
from jax.experimental.pallas import tpu as pltpu
from jax.experimental.pallas import tpu_sc as plsc

_NC = 2
_NS = 16
_NW = _NC * _NS

_BATCH = 16384
_D = 64
_PD = 2 * _D
_ENT_B = 4 * _BATCH
_EPW = _ENT_B // _NW
_CH = 512

_G = 8
_BB = _BATCH // _G
_RK = 1024


def _sc_gather_pairs(ent_pairs, ent_idx):
    mesh = plsc.VectorSubcoreMesh(core_axis_name="c", subcore_axis_name="s")

    @functools.partial(
        pl.kernel,
        out_type=jax.ShapeDtypeStruct((_ENT_B, _PD), jnp.float32),
        mesh=mesh,
        scratch_types=[
            pltpu.VMEM((_CH,), jnp.int32),
            pltpu.VMEM((_CH,), jnp.int32),
            pltpu.VMEM((_CH, _PD), jnp.float32),
            pltpu.SemaphoreType.DMA,
        ],
    )
    def k(tab_hbm, idx_hbm, out_hbm, idx_v, grp_v, pair_v, sem):
        wid = lax.axis_index("s") * _NC + lax.axis_index("c")
        wbase = wid * _EPW

        for ch in range(_EPW // _CH):
            base = wbase + ch * _CH
            pltpu.sync_copy(idx_hbm.at[pl.ds(base, _CH)], idx_v)
            for v in range(_CH // 16):
                sl = pl.ds(v * 16, 16)
                grp_v[sl] = jax.lax.shift_right_logical(idx_v[sl], 1)
            pltpu.async_copy(tab_hbm.at[grp_v], pair_v, sem).wait()
            pltpu.sync_copy(pair_v, out_hbm.at[pl.ds(base, _CH)])

    return k(ent_pairs, ent_idx)


def _tc_loss_body(ph, pt, nh, nt, phi, pti, nhi, nti, pri, nri, tab, out_ref):
    i = pl.program_id(0)

    def pick(pair, idx):
        par = (idx[...] & 1) == 1
        return jnp.where(par, pair[:, _D:], pair[:, :_D])

    h_p, t_p = pick(ph, phi), pick(pt, pti)
    h_n, t_n = pick(nh, nhi), pick(nt, nti)

    iota = jax.lax.broadcasted_iota(jnp.int32, (_BB, _RK), 1)
    oh_p = jnp.where(iota == pri[...], 1.0, 0.0).astype(jnp.float32)
    oh_n = jnp.where(iota == nri[...], 1.0, 0.0).astype(jnp.float32)
    r_p = jnp.dot(oh_p, tab[...], preferred_element_type=jnp.float32)
    r_n = jnp.dot(oh_n, tab[...], preferred_element_type=jnp.float32)

    pdiff = h_p + r_p - t_p
    ndiff = h_n + r_n - t_n
    pd2 = jnp.sum(pdiff * pdiff, axis=1) + 1e-12
    nd2 = jnp.sum(ndiff * ndiff, axis=1) + 1e-12
    part = jnp.sum(jnp.maximum(jnp.sqrt(pd2) - jnp.sqrt(nd2) + 1.0, 0.0))

    @pl.when(i == 0)
    def _():
        out_ref[0, 0] = 0.0

    out_ref[0, 0] += part

    @pl.when(i == _G - 1)
    def _():
        out_ref[0, 0] = out_ref[0, 0] * (1.0 / _BATCH)


def _tc_loss(pairs, ent_idx2, pr_idx, nr_idx, rel_pad, interpret=False):
    seg = _BATCH // _BB
    pair_spec = lambda s: pl.BlockSpec((_BB, _PD),
                                       lambda i, s=s: (s * seg + i, 0))
    eidx_spec = lambda s: pl.BlockSpec((_BB, 1),
                                       lambda i, s=s: (s * seg + i, 0))
    idx_spec = pl.BlockSpec((_BB, 1), lambda i: (i, 0))
    tab_spec = pl.BlockSpec((_RK, _D), lambda i: (0, 0))
    out = pl.pallas_call(
        _tc_loss_body,
        grid=(_G,),
        in_specs=[pair_spec(0), pair_spec(1), pair_spec(2), pair_spec(3),
                  eidx_spec(0), eidx_spec(1), eidx_spec(2), eidx_spec(3),
                  idx_spec, idx_spec, tab_spec],
        out_specs=pl.BlockSpec((1, 1), lambda i: (0, 0),
                               memory_space=pltpu.SMEM),
        out_shape=jax.ShapeDtypeStruct((1, 1), jnp.float32),
        compiler_params=pltpu.CompilerParams(
            dimension_semantics=("arbitrary",)),
        interpret=interpret,
    )(pairs, pairs, pairs, pairs,
      ent_idx2, ent_idx2, ent_idx2, ent_idx2,
      pr_idx, nr_idx, rel_pad)
    return out[0, 0]


def kernel(positive_triples, negative_triples, entities, relations):
    pt32 = positive_triples.astype(jnp.int32)
    nt32 = negative_triples.astype(jnp.int32)
    ent_idx = jnp.concatenate([pt32[:, 0], pt32[:, 2], nt32[:, 0], nt32[:, 2]])
    ent_pairs = entities.reshape(500000, _PD)
    pairs = _sc_gather_pairs(ent_pairs, ent_idx)

    return _tc_loss(
        pairs, ent_idx.reshape(_ENT_B, 1), pt32[:, 1:2], nt32[:, 1:2],
        jnp.pad(relations, ((0, _RK - relations.shape[0]), (0, 0))))


# speedup vs baseline: 2.3784x; 2.3784x over previous
import functools

import jax
import jax.numpy as jnp
from jax import lax
from jax.experimental import pallas as pl
from jax.experimental.pallas import tpu as pltpu
from jax.experimental.pallas import tpu_sc as plsc

_NC = 2
_NS = 16
_NW = _NC * _NS

_E = 1000000
_BATCH = 16384
_D = 64
_PD = 2 * _D
_ENT_B = 4 * _BATCH
_EPW = _ENT_B // _NW
_CH = 512

_PACK_W = 16384
_PACK_G = 32
_PH = 30 * 16384
_NP = _PACK_G * _PACK_W

_G = 8
_BB = _BATCH // _G
_RK = 1024


def _tc_pack_body(xl, xr, out_ref):
    eye = (jax.lax.broadcasted_iota(jnp.int32, (_D, _D), 0) ==
           jax.lax.broadcasted_iota(jnp.int32, (_D, _D), 1)).astype(jnp.bfloat16)
    dn = (((0,), (0,)), ((), ()))
    out_ref[:, :_D] = jax.lax.dot_general(
        xl[...].astype(jnp.bfloat16), eye, dn,
        preferred_element_type=jnp.float32)
    out_ref[:, _D:] = jax.lax.dot_general(
        xr[...].astype(jnp.bfloat16), eye, dn,
        preferred_element_type=jnp.float32)


def _tc_pack(ent_t, interpret=False):
    return pl.pallas_call(
        _tc_pack_body,
        grid=(_PACK_G,),
        in_specs=[
            pl.BlockSpec((_D, _PACK_W), lambda i: (0, i)),
            pl.BlockSpec((_D, _PACK_W), lambda i: (0, 30 + i)),
        ],
        out_specs=pl.BlockSpec((_PACK_W, _PD), lambda i: (i, 0)),
        out_shape=jax.ShapeDtypeStruct((_NP, _PD), jnp.float32),
        compiler_params=pltpu.CompilerParams(
            dimension_semantics=("arbitrary",),
            fuse_transposed_lhs_in_matmul=True),
        interpret=interpret,
    )(ent_t, ent_t)


def _sc_gather_pairs(pairs_tab, ent_idx):
    mesh = plsc.VectorSubcoreMesh(core_axis_name="c", subcore_axis_name="s")

    @functools.partial(
        pl.kernel,
        out_type=jax.ShapeDtypeStruct((_ENT_B, _PD), jnp.float32),
        mesh=mesh,
        scratch_types=[
            pltpu.VMEM((_CH,), jnp.int32),
            pltpu.VMEM((_CH,), jnp.int32),
            pltpu.VMEM((_CH, _PD), jnp.float32),
            pltpu.SemaphoreType.DMA,
        ],
    )
    def k(tab_hbm, idx_hbm, out_hbm, idx_v, grp_v, pair_v, sem):
        wid = lax.axis_index("s") * _NC + lax.axis_index("c")
        wbase = wid * _EPW

        for ch in range(_EPW // _CH):
            base = wbase + ch * _CH
            pltpu.sync_copy(idx_hbm.at[pl.ds(base, _CH)], idx_v)
            for v in range(_CH // 16):
                sl = pl.ds(v * 16, 16)
                i = idx_v[sl]
                grp_v[sl] = jnp.where(i >= _PH, i - _PH, i)
            pltpu.async_copy(tab_hbm.at[grp_v], pair_v, sem).wait()
            pltpu.sync_copy(pair_v, out_hbm.at[pl.ds(base, _CH)])

    return k(pairs_tab, ent_idx)


def _tc_loss_body(ph, pt, nh, nt, phi, pti, nhi, nti, pri, nri, tab, out_ref):
    i = pl.program_id(0)

    def pick(pair, idx):
        par = idx[...] >= _PH
        return jnp.where(par, pair[:, _D:],
                         pair[:, :_D]).astype(jnp.float32)

    h_p, t_p = pick(ph, phi), pick(pt, pti)
    h_n, t_n = pick(nh, nhi), pick(nt, nti)

    iota = jax.lax.broadcasted_iota(jnp.int32, (_BB, _RK), 1)
    oh_p = jnp.where(iota == pri[...], 1.0, 0.0).astype(jnp.float32)
    oh_n = jnp.where(iota == nri[...], 1.0, 0.0).astype(jnp.float32)
    r_p = jnp.dot(oh_p, tab[...], preferred_element_type=jnp.float32)
    r_n = jnp.dot(oh_n, tab[...], preferred_element_type=jnp.float32)

    pdiff = h_p + r_p - t_p
    ndiff = h_n + r_n - t_n
    pd2 = jnp.sum(pdiff * pdiff, axis=1) + 1e-12
    nd2 = jnp.sum(ndiff * ndiff, axis=1) + 1e-12
    part = jnp.sum(jnp.maximum(jnp.sqrt(pd2) - jnp.sqrt(nd2) + 1.0, 0.0))

    @pl.when(i == 0)
    def _():
        out_ref[0, 0] = 0.0

    out_ref[0, 0] += part

    @pl.when(i == _G - 1)
    def _():
        out_ref[0, 0] = out_ref[0, 0] * (1.0 / _BATCH)


def _tc_loss(pairs, ent_idx2, pr_idx, nr_idx, rel_pad, interpret=False):
    seg = _BATCH // _BB
    pair_spec = lambda s: pl.BlockSpec((_BB, _PD),
                                       lambda i, s=s: (s * seg + i, 0))
    eidx_spec = lambda s: pl.BlockSpec((_BB, 1),
                                       lambda i, s=s: (s * seg + i, 0))
    idx_spec = pl.BlockSpec((_BB, 1), lambda i: (i, 0))
    tab_spec = pl.BlockSpec((_RK, _D), lambda i: (0, 0))
    out = pl.pallas_call(
        _tc_loss_body,
        grid=(_G,),
        in_specs=[pair_spec(0), pair_spec(1), pair_spec(2), pair_spec(3),
                  eidx_spec(0), eidx_spec(1), eidx_spec(2), eidx_spec(3),
                  idx_spec, idx_spec, tab_spec],
        out_specs=pl.BlockSpec((1, 1), lambda i: (0, 0),
                               memory_space=pltpu.SMEM),
        out_shape=jax.ShapeDtypeStruct((1, 1), jnp.float32),
        compiler_params=pltpu.CompilerParams(
            dimension_semantics=("arbitrary",)),
        interpret=interpret,
    )(pairs, pairs, pairs, pairs,
      ent_idx2, ent_idx2, ent_idx2, ent_idx2,
      pr_idx, nr_idx, rel_pad)
    return out[0, 0]


def kernel(positive_triples, negative_triples, entities, relations):
    pt32 = positive_triples.astype(jnp.int32)
    nt32 = negative_triples.astype(jnp.int32)
    ent_idx = jnp.concatenate([pt32[:, 0], pt32[:, 2], nt32[:, 0], nt32[:, 2]])

    pairs_tab = _tc_pack(entities.T)
    pairs = _sc_gather_pairs(pairs_tab, ent_idx)

    return _tc_loss(
        pairs, ent_idx.reshape(_ENT_B, 1), pt32[:, 1:2], nt32[:, 1:2],
        jnp.pad(relations, ((0, _RK - relations.shape[0]), (0, 0))))
